# baseline (speedup 1.0000x reference)
import functools

import jax
import jax.numpy as jnp
from jax import lax
from jax.experimental import pallas as pl
from jax.experimental.pallas import tpu as pltpu
from jax.experimental.pallas import tpu_sc as plsc

N = 10000
NB = 166
EB = 1328
E = 640000

NC = 2
NS = 16
K = 128
NBUF = 2
WB = 64
CPT = 160
EPAD = NC * NS * K * CPT
EPC = EPAD // NC
EPT = CPT * K
RNG = 5056
RACC = 5120
SEGR = RACC // NS

P = 40
NBLK = 5000 // P
EBLK = 2048


def _relu(x):
    return jnp.maximum(x, 0.0)


def _a1_body(low_ref, srcb_ref, dstb_ref, gbn_ref, bbn_ref, wlow_ref,
             blow_ref, gm_ref, bm_ref, low_emb_ref, a_ref):
    x = low_ref[...]
    mean = jnp.mean(x, axis=0, keepdims=True)
    var = jnp.mean((x - mean) ** 2, axis=0, keepdims=True)
    xn = gbn_ref[...] * (x - mean) * lax.rsqrt(var + 1e-5) + bbn_ref[...]
    lx = _relu(jnp.dot(xn, wlow_ref[...],
                       preferred_element_type=jnp.float32) + blow_ref[...])
    m2 = jnp.mean(lx, axis=0, keepdims=True)
    v2 = jnp.mean((lx - m2) ** 2, axis=0, keepdims=True)
    low_emb_ref[...] = gm_ref[...] * (lx - m2) * lax.rsqrt(v2 + 1e-5) + bm_ref[...]

    dcol = dstb_ref[...]
    scol = srcb_ref[...]
    il = lax.broadcasted_iota(jnp.int32, (EB, NB), 1)
    ob_d = (dcol == il).astype(jnp.float32)
    ob_s = (scol == il).astype(jnp.float32)
    deg = jnp.sum(ob_d, axis=0, keepdims=True) + 1.0
    dinv = lax.rsqrt(deg)
    dv_d = jnp.sum(ob_d * dinv, axis=1, keepdims=True)
    dv_s = jnp.sum(ob_s * dinv, axis=1, keepdims=True)
    araw = lax.dot_general(ob_d * (dv_d * dv_s), ob_s,
                           (((0,), (0,)), ((), ())),
                           preferred_element_type=jnp.float32)
    ir = lax.broadcasted_iota(jnp.int32, (NB, NB), 0)
    ic = lax.broadcasted_iota(jnp.int32, (NB, NB), 1)
    a_ref[...] = araw + jnp.where(ir == ic, dinv * dinv, 0.0)


def _a2_body(x0_ref, x1_ref, x2_ref, a_ref, low_ref, w1x_ref, bt1_ref,
             w2p_ref, bt2_ref, wfgp_ref, wflp_ref, btf_ref, wc1p_ref,
             xw_ref):
    amat = a_ref[...]
    nt = (((1,), (1,)), ((), ()))
    t0 = lax.dot_general(amat, x0_ref[...], nt,
                         preferred_element_type=jnp.float32)
    t1 = lax.dot_general(amat, x1_ref[...], nt,
                         preferred_element_type=jnp.float32)
    t2 = lax.dot_general(amat, x2_ref[...], nt,
                         preferred_element_type=jnp.float32)
    tcat = jnp.concatenate([t0, t1, t2], axis=1)
    g1 = _relu(jnp.dot(tcat, w1x_ref[...],
                       preferred_element_type=jnp.float32) + bt1_ref[...])
    r4 = jnp.reshape(g1, (NB * P, 128))
    r5 = jnp.dot(r4, w2p_ref[...], preferred_element_type=jnp.float32)
    r6 = jnp.reshape(r5, (NB, P * 128))
    g2 = _relu(jnp.dot(amat, r6, preferred_element_type=jnp.float32)
               + bt2_ref[...])
    s = jnp.sum(g2, axis=0, keepdims=True)
    gg = jnp.reshape(s, (P, 128))
    h = _relu(jnp.dot(gg, wfgp_ref[...], preferred_element_type=jnp.float32)
              + jnp.dot(low_ref[...], wflp_ref[...],
                        preferred_element_type=jnp.float32)
              + btf_ref[...])
    xw_ref[...] = jnp.dot(h, wc1p_ref[...], preferred_element_type=jnp.float32)


def _h_body(d_ref, deg_ref):
    i = pl.program_id(0)
    d = d_ref[...]
    hi = lax.shift_right_logical(d, 7)
    lo = jnp.bitwise_and(d, 127)
    ih = lax.broadcasted_iota(jnp.int32, (EBLK, 80), 1)
    il = lax.broadcasted_iota(jnp.int32, (EBLK, 128), 1)
    ohh = (hi == ih).astype(jnp.float32)
    ohl = (lo == il).astype(jnp.float32)
    blk = lax.dot_general(ohh, ohl, (((0,), (0,)), ((), ())),
                          preferred_element_type=jnp.float32)

    @pl.when(i == 0)
    def _():
        deg_ref[...] = jnp.zeros_like(deg_ref)

    deg_ref[...] += blk


def _b_body(deg_ref, xw_ref, y_ref, dinv_ref):
    deg = deg_ref[...] + 1.0
    dinv = lax.rsqrt(deg)
    xw = xw_ref[...]
    y_ref[:, 0:64] = xw * dinv
    y_ref[:, 64:128] = jnp.zeros((N, 64), jnp.float32)
    dinv_ref[...] = jnp.broadcast_to(dinv, (N, 64))


def _c_body(p0_ref, p1_ref, dinv_ref, xw_ref, bc1_ref, wcls_ref, bcls_ref,
            out_ref):
    dinv = dinv_ref[...]
    conv = (dinv * (p0_ref[...] + p1_ref[...])
            + dinv * dinv * xw_ref[...] + bc1_ref[...])
    t = jnp.tanh(conv)
    logits = jnp.dot(t, wcls_ref[...],
                     preferred_element_type=jnp.float32) + bcls_ref[...]
    m = jnp.max(logits, axis=1, keepdims=True)
    lse = jnp.log(jnp.sum(jnp.exp(logits - m), axis=1, keepdims=True)) + m
    out_ref[...] = logits - lse


def _s2_body(src_hbm, dst_hbm, y_hbm, zero_hbm, out_hbm, *scratch):
    si = scratch[0:NBUF]
    di = scratch[NBUF:2 * NBUF]
    dj = scratch[2 * NBUF:3 * NBUF]
    rows = scratch[3 * NBUF:4 * NBUF]
    buf_v = scratch[4 * NBUF]
    acc_sh = scratch[4 * NBUF + 1]
    sem = scratch[4 * NBUF + 2]
    c = lax.axis_index("c")
    s = lax.axis_index("s")
    base = c * EPC + s * EPT

    for p in range(2):
        lo = p * RNG
        hi = lo + RNG
        pltpu.sync_copy(zero_hbm, buf_v)
        for i in range(SEGR // WB):
            pltpu.sync_copy(buf_v, acc_sh.at[pl.ds(s * SEGR + i * WB, WB)])
        plsc.subcore_barrier()

        def group(g, carry):
            offs = [base + (g * NBUF + b) * K for b in range(NBUF)]
            ds_ = [pltpu.async_copy(src_hbm.at[pl.ds(offs[b], K)], si[b], sem)
                   for b in range(NBUF)]
            ds_ += [pltpu.async_copy(dst_hbm.at[pl.ds(offs[b], K)], di[b], sem)
                    for b in range(NBUF)]
            for d in ds_:
                d.wait()
            for b in range(NBUF):
                for q in range(K // 16):
                    d16 = di[b][pl.ds(q * 16, 16)]
                    m = jnp.logical_and(d16 >= lo, d16 < hi)
                    dj[b][pl.ds(q * 16, 16)] = jnp.where(m, d16 - lo, RNG)
            gs = [pltpu.async_copy(y_hbm.at[si[b]], rows[b], sem)
                  for b in range(NBUF)]
            for d in gs:
                d.wait()
            ss = [pltpu.async_copy(rows[b], acc_sh.at[dj[b]], sem, add=True)
                  for b in range(NBUF)]
            for d in ss:
                d.wait()
            return carry

        lax.fori_loop(0, CPT // NBUF, group, 0)
        plsc.subcore_barrier()

        for i in range(SEGR // WB):
            pltpu.sync_copy(acc_sh.at[pl.ds(s * SEGR + i * WB, WB)], buf_v)
            pltpu.sync_copy(buf_v, out_hbm.at[
                pl.ds((c * 2 + p) * RACC + s * SEGR + i * WB, WB)])
        plsc.subcore_barrier()


def _blockdiag2(w):
    z = jnp.zeros_like(w)
    return jnp.concatenate(
        [jnp.concatenate([w, z], axis=1), jnp.concatenate([z, w], axis=1)],
        axis=0)


def kernel(high_dim_features, low_dim_features, brain_edge_index, edge_index,
           gamma_bnlow, beta_bnlow, W_low, b_low, gamma_mlplow, beta_mlplow,
           W_g1, b_g1, W_g2, b_g2, W_fc1, b_fc1, W_c1, b_c1, W_cls, b_cls):
    f32 = jnp.float32
    x3 = high_dim_features.reshape(N, NB, 3)
    x0 = x3[:, :, 0]
    x1 = x3[:, :, 1]
    x2 = x3[:, :, 2]
    srcb = brain_edge_index[0].reshape(EB, 1)
    dstb = brain_edge_index[1].reshape(EB, 1)
    src = edge_index[0]
    dst = edge_index[1]
    src_p = jnp.concatenate([src, jnp.zeros((EPAD - E,), jnp.int32)])
    dst_p = jnp.concatenate([dst, jnp.full((EPAD - E,), N, jnp.int32)])

    gbn = gamma_bnlow.reshape(1, 64)
    bbn = beta_bnlow.reshape(1, 64)
    blow = b_low.reshape(1, 64)
    gm = gamma_mlplow.reshape(1, 64)
    bm = beta_mlplow.reshape(1, 64)

    bidx = jnp.arange(2 * P)
    ohp = (bidx[:, None] // 2 == jnp.arange(P)[None, :]).astype(f32)
    w4 = ohp[None, :, :, None] * W_g1[:, None, None, :]
    s_of = (bidx % 2).astype(f32)[None, :, None, None]
    w1x = jnp.concatenate([w4 * (1.0 - s_of), w4 * s_of],
                          axis=3).reshape(3 * 2 * P, P * 128)
    b1pair = jnp.concatenate([b_g1, b_g1])
    bt1 = jnp.tile(b1pair, P).reshape(1, P * 128)
    w2p = _blockdiag2(W_g2)
    bt2 = jnp.tile(jnp.concatenate([b_g2, b_g2]), P).reshape(1, P * 128)
    wfgp = _blockdiag2(W_fc1[:64]) / jnp.asarray(NB, f32)
    wflp = _blockdiag2(W_fc1[64:])
    btf = jnp.concatenate([b_fc1, b_fc1]).reshape(1, 128)
    wc1p = _blockdiag2(W_c1)
    bc1 = b_c1.reshape(1, 64)
    bcls = b_cls.reshape(1, 10)

    low_emb, a_br = pl.pallas_call(
        _a1_body,
        out_shape=[jax.ShapeDtypeStruct((N, 64), f32),
                   jax.ShapeDtypeStruct((NB, NB), f32)],
    )(low_dim_features, srcb, dstb, gbn, bbn, W_low, blow, gm, bm)

    low_pair = low_emb.reshape(N // 2, 128)

    full = lambda shape: pl.BlockSpec(shape, lambda i: tuple(0 for _ in shape))
    xw_pair = pl.pallas_call(
        _a2_body,
        grid=(NBLK,),
        in_specs=[
            pl.BlockSpec((2 * P, NB), lambda i: (i, 0)),
            pl.BlockSpec((2 * P, NB), lambda i: (i, 0)),
            pl.BlockSpec((2 * P, NB), lambda i: (i, 0)),
            full((NB, NB)),
            pl.BlockSpec((P, 128), lambda i: (i, 0)),
            full((6 * P, P * 128)), full((1, P * 128)), full((128, 128)),
            full((1, P * 128)), full((128, 128)), full((128, 128)),
            full((1, 128)), full((128, 128)),
        ],
        out_specs=pl.BlockSpec((P, 128), lambda i: (i, 0)),
        out_shape=jax.ShapeDtypeStruct((N // 2, 128), f32),
    )(x0, x1, x2, a_br, low_pair, w1x, bt1, w2p, bt2, wfgp, wflp, btf, wc1p)
    xw = xw_pair.reshape(N, 64)

    deg2d = pl.pallas_call(
        _h_body,
        grid=(EPAD // EBLK,),
        in_specs=[pl.BlockSpec((EBLK, 1), lambda i: (i, 0))],
        out_specs=pl.BlockSpec((80, 128), lambda i: (0, 0)),
        out_shape=jax.ShapeDtypeStruct((80, 128), f32),
    )(dst_p.reshape(EPAD, 1))
    deg_col = deg2d.reshape(80 * 128, 1)[:N]

    y, dinv64 = pl.pallas_call(
        _b_body,
        out_shape=[jax.ShapeDtypeStruct((N, 128), f32),
                   jax.ShapeDtypeStruct((N, 64), f32)],
    )(deg_col, xw)

    mesh = plsc.VectorSubcoreMesh(core_axis_name="c", subcore_axis_name="s",
                                  num_cores=NC, num_subcores=NS)
    zrows = jnp.zeros((WB, 128), f32)
    part = pl.kernel(
        _s2_body,
        out_type=jax.ShapeDtypeStruct((NC * 2 * RACC, 128), f32),
        mesh=mesh,
        scratch_types=(
            [pltpu.VMEM((K,), jnp.int32) for _ in range(3 * NBUF)]
            + [pltpu.VMEM((K, 128), f32) for _ in range(NBUF)]
            + [pltpu.VMEM((WB, 128), f32),
               pltpu.VMEM_SHARED((RACC, 128), f32),
               pltpu.SemaphoreType.DMA]
        ),
    )(src_p, dst_p, y, zrows)
    o4 = part.reshape(NC, 2, RACC, 128)
    p0 = o4[0, :, :RNG].reshape(2 * RNG, 128)[:N, :64]
    p1 = o4[1, :, :RNG].reshape(2 * RNG, 128)[:N, :64]

    out = pl.pallas_call(
        _c_body,
        out_shape=jax.ShapeDtypeStruct((N, 10), f32),
    )(p0, p1, dinv64, xw, bc1, W_cls, bcls)
    return out


# speedup vs baseline: 1.2992x; 1.2992x over previous
import functools

import jax
import jax.numpy as jnp
from jax import lax
from jax.experimental import pallas as pl
from jax.experimental.pallas import tpu as pltpu
from jax.experimental.pallas import tpu_sc as plsc

N = 10000
NB = 166
EB = 1328
E = 640000

NC = 2
NS = 16
K = 128
CPT = 157
EPAD = NC * NS * K * CPT
EPC = EPAD // NC
EPT = CPT * K
RNG = 5056
RACC = 5120
SEGR = RACC // NS

P = 40
NBLK = 5000 // P
EBLK = 2048


def _relu(x):
    return jnp.maximum(x, 0.0)


def _a1_body(low_ref, srcb_ref, dstb_ref, gbn_ref, bbn_ref, wlow_ref,
             blow_ref, gm_ref, bm_ref, low_emb_ref, a_ref):
    x = low_ref[...]
    mean = jnp.mean(x, axis=0, keepdims=True)
    var = jnp.mean((x - mean) ** 2, axis=0, keepdims=True)
    xn = gbn_ref[...] * (x - mean) * lax.rsqrt(var + 1e-5) + bbn_ref[...]
    lx = _relu(jnp.dot(xn, wlow_ref[...],
                       preferred_element_type=jnp.float32) + blow_ref[...])
    m2 = jnp.mean(lx, axis=0, keepdims=True)
    v2 = jnp.mean((lx - m2) ** 2, axis=0, keepdims=True)
    low_emb_ref[...] = gm_ref[...] * (lx - m2) * lax.rsqrt(v2 + 1e-5) + bm_ref[...]

    dcol = dstb_ref[...]
    scol = srcb_ref[...]
    il = lax.broadcasted_iota(jnp.int32, (EB, NB), 1)
    ob_d = (dcol == il).astype(jnp.float32)
    ob_s = (scol == il).astype(jnp.float32)
    deg = jnp.sum(ob_d, axis=0, keepdims=True) + 1.0
    dinv = lax.rsqrt(deg)
    dv_d = jnp.sum(ob_d * dinv, axis=1, keepdims=True)
    dv_s = jnp.sum(ob_s * dinv, axis=1, keepdims=True)
    araw = lax.dot_general(ob_d * (dv_d * dv_s), ob_s,
                           (((0,), (0,)), ((), ())),
                           preferred_element_type=jnp.float32)
    ir = lax.broadcasted_iota(jnp.int32, (NB, NB), 0)
    ic = lax.broadcasted_iota(jnp.int32, (NB, NB), 1)
    a_ref[...] = araw + jnp.where(ir == ic, dinv * dinv, 0.0)


def _a2_body(x0_ref, x1_ref, x2_ref, a_ref, low_ref, w1x_ref, bt1_ref,
             w2p_ref, bt2_ref, wfgp_ref, wflp_ref, btf_ref, wc1p_ref,
             xw_ref):
    amat = a_ref[...]
    nt = (((1,), (1,)), ((), ()))
    t0 = lax.dot_general(amat, x0_ref[...], nt,
                         preferred_element_type=jnp.float32)
    t1 = lax.dot_general(amat, x1_ref[...], nt,
                         preferred_element_type=jnp.float32)
    t2 = lax.dot_general(amat, x2_ref[...], nt,
                         preferred_element_type=jnp.float32)
    tcat = jnp.concatenate([t0, t1, t2], axis=1)
    g1 = _relu(jnp.dot(tcat, w1x_ref[...],
                       preferred_element_type=jnp.float32) + bt1_ref[...])
    r4 = jnp.reshape(g1, (NB * P, 128))
    r5 = jnp.dot(r4, w2p_ref[...], preferred_element_type=jnp.float32)
    r6 = jnp.reshape(r5, (NB, P * 128))
    g2 = _relu(jnp.dot(amat, r6, preferred_element_type=jnp.float32)
               + bt2_ref[...])
    s = jnp.sum(g2, axis=0, keepdims=True)
    gg = jnp.reshape(s, (P, 128))
    h = _relu(jnp.dot(gg, wfgp_ref[...], preferred_element_type=jnp.float32)
              + jnp.dot(low_ref[...], wflp_ref[...],
                        preferred_element_type=jnp.float32)
              + btf_ref[...])
    xw_ref[...] = jnp.dot(h, wc1p_ref[...], preferred_element_type=jnp.float32)


def _h_body(d_ref, deg_ref):
    i = pl.program_id(0)
    d = d_ref[...]
    hi = lax.shift_right_logical(d, 7)
    lo = jnp.bitwise_and(d, 127)
    ih = lax.broadcasted_iota(jnp.int32, (EBLK, 80), 1)
    il = lax.broadcasted_iota(jnp.int32, (EBLK, 128), 1)
    ohh = (hi == ih).astype(jnp.float32)
    ohl = (lo == il).astype(jnp.float32)
    blk = lax.dot_general(ohh, ohl, (((0,), (0,)), ((), ())),
                          preferred_element_type=jnp.float32)

    @pl.when(i == 0)
    def _():
        deg_ref[...] = jnp.zeros_like(deg_ref)

    deg_ref[...] += blk


def _b_body(deg_ref, xw_ref, y_ref, dinv_ref):
    deg = deg_ref[...] + 1.0
    dinv = lax.rsqrt(deg)
    xw = xw_ref[...]
    y_ref[:, 0:64] = xw * dinv
    y_ref[:, 64:128] = jnp.zeros((N, 64), jnp.float32)
    dinv_ref[...] = jnp.broadcast_to(dinv, (N, 64))


def _c_body(p0_ref, p1_ref, dinv_ref, xw_ref, bc1_ref, wcls_ref, bcls_ref,
            out_ref):
    dinv = dinv_ref[...]
    conv = (dinv * (p0_ref[...] + p1_ref[...])
            + dinv * dinv * xw_ref[...] + bc1_ref[...])
    t = jnp.tanh(conv)
    logits = jnp.dot(t, wcls_ref[...],
                     preferred_element_type=jnp.float32) + bcls_ref[...]
    m = jnp.max(logits, axis=1, keepdims=True)
    lse = jnp.log(jnp.sum(jnp.exp(logits - m), axis=1, keepdims=True)) + m
    out_ref[...] = logits - lse


def _s2_body(src_hbm, dst_hbm, y_hbm, zero_hbm, out_hbm, si_v, di_v, dj_v,
             rows_v, buf_v, acc_sh, sem):
    c = lax.axis_index("c")
    s = lax.axis_index("s")
    base = c * EPC + s * EPT

    for p in range(2):
        lo = p * RNG
        hi = lo + RNG
        pltpu.sync_copy(zero_hbm, buf_v)
        pltpu.sync_copy(buf_v, acc_sh.at[pl.ds(s * SEGR, SEGR)])
        plsc.subcore_barrier()

        def chunk(j, carry):
            pltpu.sync_copy(src_hbm.at[pl.ds(base + j * K, K)], si_v)
            pltpu.sync_copy(dst_hbm.at[pl.ds(base + j * K, K)], di_v)
            for q in range(K // 16):
                d16 = di_v[pl.ds(q * 16, 16)]
                m = jnp.logical_and(d16 >= lo, d16 < hi)
                dj_v[pl.ds(q * 16, 16)] = jnp.where(m, d16 - lo, RNG)
            pltpu.async_copy(y_hbm.at[si_v], rows_v, sem).wait()
            pltpu.sync_copy(rows_v, acc_sh.at[dj_v], add=True)
            return carry

        lax.fori_loop(0, CPT, chunk, 0)
        plsc.subcore_barrier()

        pltpu.sync_copy(acc_sh.at[pl.ds(s * SEGR, SEGR)], buf_v)
        pltpu.sync_copy(
            buf_v, out_hbm.at[pl.ds((c * 2 + p) * RACC + s * SEGR, SEGR)])
        plsc.subcore_barrier()


def _blockdiag2(w):
    z = jnp.zeros_like(w)
    return jnp.concatenate(
        [jnp.concatenate([w, z], axis=1), jnp.concatenate([z, w], axis=1)],
        axis=0)


def kernel(high_dim_features, low_dim_features, brain_edge_index, edge_index,
           gamma_bnlow, beta_bnlow, W_low, b_low, gamma_mlplow, beta_mlplow,
           W_g1, b_g1, W_g2, b_g2, W_fc1, b_fc1, W_c1, b_c1, W_cls, b_cls):
    f32 = jnp.float32
    x3 = high_dim_features.reshape(N, NB, 3)
    x0 = x3[:, :, 0]
    x1 = x3[:, :, 1]
    x2 = x3[:, :, 2]
    srcb = brain_edge_index[0].reshape(EB, 1)
    dstb = brain_edge_index[1].reshape(EB, 1)
    src = edge_index[0]
    dst = edge_index[1]
    src_p = jnp.concatenate([src, jnp.zeros((EPAD - E,), jnp.int32)])
    dst_p = jnp.concatenate([dst, jnp.full((EPAD - E,), N, jnp.int32)])

    gbn = gamma_bnlow.reshape(1, 64)
    bbn = beta_bnlow.reshape(1, 64)
    blow = b_low.reshape(1, 64)
    gm = gamma_mlplow.reshape(1, 64)
    bm = beta_mlplow.reshape(1, 64)

    bidx = jnp.arange(2 * P)
    ohp = (bidx[:, None] // 2 == jnp.arange(P)[None, :]).astype(f32)
    w4 = ohp[None, :, :, None] * W_g1[:, None, None, :]
    s_of = (bidx % 2).astype(f32)[None, :, None, None]
    w1x = jnp.concatenate([w4 * (1.0 - s_of), w4 * s_of],
                          axis=3).reshape(3 * 2 * P, P * 128)
    b1pair = jnp.concatenate([b_g1, b_g1])
    bt1 = jnp.tile(b1pair, P).reshape(1, P * 128)
    w2p = _blockdiag2(W_g2)
    bt2 = jnp.tile(jnp.concatenate([b_g2, b_g2]), P).reshape(1, P * 128)
    wfgp = _blockdiag2(W_fc1[:64]) / jnp.asarray(NB, f32)
    wflp = _blockdiag2(W_fc1[64:])
    btf = jnp.concatenate([b_fc1, b_fc1]).reshape(1, 128)
    wc1p = _blockdiag2(W_c1)
    bc1 = b_c1.reshape(1, 64)
    bcls = b_cls.reshape(1, 10)

    low_emb, a_br = pl.pallas_call(
        _a1_body,
        out_shape=[jax.ShapeDtypeStruct((N, 64), f32),
                   jax.ShapeDtypeStruct((NB, NB), f32)],
    )(low_dim_features, srcb, dstb, gbn, bbn, W_low, blow, gm, bm)

    low_pair = low_emb.reshape(N // 2, 128)

    full = lambda shape: pl.BlockSpec(shape, lambda i: tuple(0 for _ in shape))
    xw_pair = pl.pallas_call(
        _a2_body,
        grid=(NBLK,),
        in_specs=[
            pl.BlockSpec((2 * P, NB), lambda i: (i, 0)),
            pl.BlockSpec((2 * P, NB), lambda i: (i, 0)),
            pl.BlockSpec((2 * P, NB), lambda i: (i, 0)),
            full((NB, NB)),
            pl.BlockSpec((P, 128), lambda i: (i, 0)),
            full((6 * P, P * 128)), full((1, P * 128)), full((128, 128)),
            full((1, P * 128)), full((128, 128)), full((128, 128)),
            full((1, 128)), full((128, 128)),
        ],
        out_specs=pl.BlockSpec((P, 128), lambda i: (i, 0)),
        out_shape=jax.ShapeDtypeStruct((N // 2, 128), f32),
    )(x0, x1, x2, a_br, low_pair, w1x, bt1, w2p, bt2, wfgp, wflp, btf, wc1p)
    xw = xw_pair.reshape(N, 64)

    deg2d = pl.pallas_call(
        _h_body,
        grid=(EPAD // EBLK,),
        in_specs=[pl.BlockSpec((EBLK, 1), lambda i: (i, 0))],
        out_specs=pl.BlockSpec((80, 128), lambda i: (0, 0)),
        out_shape=jax.ShapeDtypeStruct((80, 128), f32),
    )(dst_p.reshape(EPAD, 1))
    deg_col = deg2d.reshape(80 * 128, 1)[:N]

    y, dinv64 = pl.pallas_call(
        _b_body,
        out_shape=[jax.ShapeDtypeStruct((N, 128), f32),
                   jax.ShapeDtypeStruct((N, 64), f32)],
    )(deg_col, xw)

    mesh = plsc.VectorSubcoreMesh(core_axis_name="c", subcore_axis_name="s",
                                  num_cores=NC, num_subcores=NS)
    zrows = jnp.zeros((SEGR, 128), f32)
    part = pl.kernel(
        _s2_body,
        out_type=jax.ShapeDtypeStruct((NC * 2 * RACC, 128), f32),
        mesh=mesh,
        scratch_types=[
            pltpu.VMEM((K,), jnp.int32),
            pltpu.VMEM((K,), jnp.int32),
            pltpu.VMEM((K,), jnp.int32),
            pltpu.VMEM((K, 128), f32),
            pltpu.VMEM((SEGR, 128), f32),
            pltpu.VMEM_SHARED((RACC, 128), f32),
            pltpu.SemaphoreType.DMA,
        ],
    )(src_p, dst_p, y, zrows)
    o4 = part.reshape(NC, 2, RACC, 128)
    p0 = o4[0, :, :RNG].reshape(2 * RNG, 128)[:N, :64]
    p1 = o4[1, :, :RNG].reshape(2 * RNG, 128)[:N, :64]

    out = pl.pallas_call(
        _c_body,
        out_shape=jax.ShapeDtypeStruct((N, 10), f32),
    )(p0, p1, dinv64, xw, bc1, W_cls, bcls)
    return out
